# baseline (speedup 1.0000x reference)
import functools

import jax
import jax.numpy as jnp
from jax import lax
from jax.experimental import pallas as pl
from jax.experimental.pallas import tpu as pltpu


def _phase_kernel(x_ref, w_ref, b_ref, o_ref, xcol_ref, ybuf_ref, sems,
                  *, H, W, C, th):
    W2 = 2 * W
    pix = th * W
    n_bands = H // th
    PW = (H + 2) * W

    xt = jnp.transpose(x_ref[0].astype(jnp.bfloat16))
    zrow = jnp.zeros((W, C), jnp.bfloat16)
    xtp = jnp.concatenate([zrow, xt, zrow], axis=0)
    iota = lax.broadcasted_iota(jnp.int32, (PW, 1), 0)
    z1 = jnp.zeros((1, C), jnp.bfloat16)
    xl = jnp.where(iota % W == 0, 0,
                   jnp.concatenate([z1, xtp[:-1]], axis=0))
    xr = jnp.where(iota % W == W - 1, 0,
                   jnp.concatenate([xtp[1:], z1], axis=0))
    xcol_ref[...] = jnp.concatenate([xl, xtp, xr], axis=1)

    bias0 = b_ref[...]
    S = 2 * th * W2
    n = pl.program_id(0)
    DEPTH = 4
    dmas = [None] * DEPTH
    for k in range(n_bands):
        base = (1 + k * th) * W
        ys = []
        for di in range(2):
            acc = jnp.zeros((pix, 2 * C), jnp.float32) + bias0
            for ty in range(2):
                s = base + (di + ty - 1) * W
                acc = acc + jnp.dot(
                    xcol_ref[s:s + pix], w_ref[di * 2 + ty],
                    preferred_element_type=jnp.float32)
            ys.append(acc.reshape(2 * pix, C).reshape(th, W2, C))
        y = jnp.stack(ys, axis=1).reshape(S, C)
        slot = k % DEPTH
        if dmas[slot] is not None:
            dmas[slot].wait()
        ybuf_ref[slot] = jnp.transpose(y.astype(ybuf_ref.dtype))
        dmas[slot] = pltpu.make_async_copy(
            ybuf_ref.at[slot], o_ref.at[n, :, k * S:(k + 1) * S],
            sems.at[slot])
        dmas[slot].start()
    for d in dmas:
        if d is not None:
            d.wait()


def kernel(x, weight, bias):
    N, C, H, W = x.shape
    H2, W2 = 2 * H, 2 * W
    th = next(t for t in (8, 4, 2, 1) if H % t == 0)
    n_bands = H // th

    w33 = jnp.transpose(weight, (2, 3, 1, 0))
    R = jnp.array([[[1, 0, 0], [0, 1, 1]],
                   [[1, 1, 0], [0, 0, 1]]], jnp.float32)
    wc = jnp.einsum('ayh,bxw,hwio->abyxio', R, R, w33)
    z = jnp.zeros((2, 2, C, C), jnp.float32)
    slot0 = jnp.concatenate([wc[:, 0, :, 0], z], axis=3)
    slot1 = jnp.concatenate([wc[:, 0, :, 1], wc[:, 1, :, 0]], axis=3)
    slot2 = jnp.concatenate([z, wc[:, 1, :, 1]], axis=3)
    w3 = jnp.concatenate([slot0, slot1, slot2], axis=2)
    w3 = w3.reshape(4, 3 * C, 2 * C).astype(jnp.bfloat16)
    b2 = jnp.tile(bias.reshape(1, C), (1, 2)).astype(jnp.float32)

    out = pl.pallas_call(
        functools.partial(_phase_kernel, H=H, W=W, C=C, th=th),
        out_shape=jax.ShapeDtypeStruct((N, C, H2 * W2), x.dtype),
        grid_spec=pltpu.PrefetchScalarGridSpec(
            num_scalar_prefetch=0,
            grid=(N,),
            in_specs=[
                pl.BlockSpec((1, C, H * W), lambda n: (n, 0, 0)),
                pl.BlockSpec((4, 3 * C, 2 * C), lambda n: (0, 0, 0)),
                pl.BlockSpec((1, 2 * C), lambda n: (0, 0)),
            ],
            out_specs=pl.BlockSpec(memory_space=pl.ANY),
            scratch_shapes=[
                pltpu.VMEM(((H + 2) * W, 3 * C), jnp.bfloat16),
                pltpu.VMEM((4, C, 2 * th * 2 * W), x.dtype),
                pltpu.SemaphoreType.DMA((4,)),
            ],
        ),
        compiler_params=pltpu.CompilerParams(
            dimension_semantics=("parallel",),
            vmem_limit_bytes=100 << 20),
    )(x.reshape(N, C, H * W), w3, b2)
    return out.reshape(N, C, H2, W2)


# speedup vs baseline: 1.0891x; 1.0891x over previous
import functools

import jax
import jax.numpy as jnp
from jax import lax
from jax.experimental import pallas as pl
from jax.experimental.pallas import tpu as pltpu


def _phase_kernel(x_ref, w_ref, b_ref, o_ref, xcol_ref, *, H, W, C, th):
    W2 = 2 * W
    pix = th * W
    n_bands = H // th
    PW = (H + 2) * W

    xt = jnp.transpose(x_ref[0].astype(jnp.bfloat16))
    zrow = jnp.zeros((W, C), jnp.bfloat16)
    xtp = jnp.concatenate([zrow, xt, zrow], axis=0)
    iota = lax.broadcasted_iota(jnp.int32, (PW, 1), 0)
    z1 = jnp.zeros((1, C), jnp.bfloat16)
    xl = jnp.where(iota % W == 0, 0,
                   jnp.concatenate([z1, xtp[:-1]], axis=0))
    xr = jnp.where(iota % W == W - 1, 0,
                   jnp.concatenate([xtp[1:], z1], axis=0))
    xcol_ref[...] = jnp.concatenate([xl, xtp, xr], axis=1)

    bias0 = b_ref[...]
    S = 2 * th * W2
    for k in range(n_bands):
        base = (1 + k * th) * W
        ys = []
        for di in range(2):
            acc = jnp.zeros((pix, 2 * C), jnp.float32) + bias0
            for ty in range(2):
                s = base + (di + ty - 1) * W
                acc = acc + jnp.dot(
                    xcol_ref[s:s + pix], w_ref[di * 2 + ty],
                    preferred_element_type=jnp.float32)
            ys.append(acc.reshape(2 * pix, C).reshape(th, W2, C))
        y = jnp.stack(ys, axis=1).reshape(S, C)
        o_ref[0, :, k * S:(k + 1) * S] = jnp.transpose(y.astype(o_ref.dtype))


def kernel(x, weight, bias):
    N, C, H, W = x.shape
    H2, W2 = 2 * H, 2 * W
    th = next(t for t in (8, 4, 2, 1) if H % t == 0)
    n_bands = H // th

    w33 = jnp.transpose(weight, (2, 3, 1, 0))
    R = jnp.array([[[1, 0, 0], [0, 1, 1]],
                   [[1, 1, 0], [0, 0, 1]]], jnp.float32)
    wc = jnp.einsum('ayh,bxw,hwio->abyxio', R, R, w33)
    z = jnp.zeros((2, 2, C, C), jnp.float32)
    slot0 = jnp.concatenate([wc[:, 0, :, 0], z], axis=3)
    slot1 = jnp.concatenate([wc[:, 0, :, 1], wc[:, 1, :, 0]], axis=3)
    slot2 = jnp.concatenate([z, wc[:, 1, :, 1]], axis=3)
    w3 = jnp.concatenate([slot0, slot1, slot2], axis=2)
    w3 = w3.reshape(4, 3 * C, 2 * C).astype(jnp.bfloat16)
    b2 = jnp.tile(bias.reshape(1, C), (1, 2)).astype(jnp.float32)

    out = pl.pallas_call(
        functools.partial(_phase_kernel, H=H, W=W, C=C, th=th),
        out_shape=jax.ShapeDtypeStruct((N, C, H2 * W2), x.dtype),
        grid_spec=pltpu.PrefetchScalarGridSpec(
            num_scalar_prefetch=0,
            grid=(N,),
            in_specs=[
                pl.BlockSpec((1, C, H * W), lambda n: (n, 0, 0)),
                pl.BlockSpec((4, 3 * C, 2 * C), lambda n: (0, 0, 0)),
                pl.BlockSpec((1, 2 * C), lambda n: (0, 0)),
            ],
            out_specs=pl.BlockSpec((1, C, H2 * W2), lambda n: (n, 0, 0)),
            scratch_shapes=[
                pltpu.VMEM(((H + 2) * W, 3 * C), jnp.bfloat16),
            ],
        ),
        compiler_params=pltpu.CompilerParams(
            dimension_semantics=("parallel",),
            vmem_limit_bytes=100 << 20),
    )(x.reshape(N, C, H * W), w3, b2)
    return out.reshape(N, C, H2, W2)
